# baseline (speedup 1.0000x reference)
import numpy as np
import jax
import jax.numpy as jnp
from jax.experimental import pallas as pl
from jax.experimental.pallas import tpu as pltpu

ALPHA = 4


def _slow_indices(T: int):
    return [int(v) for v in np.linspace(0, T - 1, T // ALPHA).astype(np.int64)]


def _make_dma_body(idx, C, T, FAST_CHUNKS):
    S = len(idx)

    def _dma_body(src_ref, slow_ref, fast_ref, sem_fast, sem_slow):
        fast_copies = []
        step = T // FAST_CHUNKS
        for c in range(C):
            for j in range(FAST_CHUNKS):
                cp = pltpu.make_async_copy(
                    src_ref.at[c, pl.ds(j * step, step)],
                    fast_ref.at[c, pl.ds(j * step, step)],
                    sem_fast,
                )
                cp.start()
                fast_copies.append(cp)
        slow_copies = []
        for c in range(C):
            for k in range(S):
                cp = pltpu.make_async_copy(
                    src_ref.at[c, pl.ds(idx[k], 1)],
                    slow_ref.at[c, pl.ds(k, 1)],
                    sem_slow,
                )
                cp.start()
                slow_copies.append(cp)
        for cp in fast_copies:
            cp.wait()
        for cp in slow_copies:
            cp.wait()

    return _dma_body


def kernel(frames):
    C, T, H, W = frames.shape
    idx = _slow_indices(T)
    S = len(idx)

    slow, fast = pl.pallas_call(
        _make_dma_body(idx, C, T, FAST_CHUNKS=8),
        in_specs=[pl.BlockSpec(memory_space=pltpu.MemorySpace.HBM)],
        out_specs=[
            pl.BlockSpec(memory_space=pltpu.MemorySpace.HBM),
            pl.BlockSpec(memory_space=pltpu.MemorySpace.HBM),
        ],
        out_shape=[
            jax.ShapeDtypeStruct((C, S, H, W), frames.dtype),
            jax.ShapeDtypeStruct((C, T, H, W), frames.dtype),
        ],
        scratch_shapes=[pltpu.SemaphoreType.DMA, pltpu.SemaphoreType.DMA],
    )(frames)

    return (slow, fast)
